# baseline (speedup 1.0000x reference)
import functools

import jax
import jax.numpy as jnp
from jax import lax
from jax.experimental import pallas as pl
from jax.experimental.pallas import tpu as pltpu
from jax.experimental.pallas import tpu_sc as plsc

EMB = 32
OUT = 128
BATCH = 16384
SEQ = 200
VOCAB = 1000000
RBLK = 16384
NRBLK = -(-VOCAB // RBLK)
VOCAB_PAD = NRBLK * RBLK

NC = 2
NS = 16
NW = NC * NS
ROWS_W = BATCH // NW
IDX_W = ROWS_W * SEQ
CHUNK = 800
RPC = CHUNK // SEQ
NCH = IDX_W // CHUNK
NBUF = 4


def _tc_repack_table(tT):
    def body(t_ref, o_ref):
        x = t_ref[...]
        q = RBLK // 4
        stacked = jnp.concatenate(
            [x[:, j * q:(j + 1) * q] for j in range(4)], axis=0)
        o_ref[...] = jnp.transpose(stacked)

    return pl.pallas_call(
        body,
        out_shape=jax.ShapeDtypeStruct((VOCAB_PAD // 4, 128), jnp.float32),
        grid=(NRBLK,),
        in_specs=[pl.BlockSpec((EMB, RBLK), lambda i: (0, i))],
        out_specs=pl.BlockSpec((RBLK // 4, 128), lambda i: (i, 0)),
    )(tT)


def _sc_pool_sums(x4, seg4, zeros, table):
    mesh = plsc.VectorSubcoreMesh(core_axis_name="c", subcore_axis_name="s")

    scratch = (
        [pltpu.VMEM((CHUNK,), jnp.int32) for _ in range(NBUF)]
        + [pltpu.VMEM((CHUNK, EMB), jnp.float32) for _ in range(NBUF)]
        + [pltpu.VMEM((CHUNK,), jnp.int32)]
        + [pltpu.VMEM_SHARED((NS * ROWS_W, EMB), jnp.float32)]
        + [pltpu.SemaphoreType.DMA for _ in range(3 * NBUF)]
    )

    @functools.partial(
        pl.kernel,
        out_type=jax.ShapeDtypeStruct((BATCH, EMB), jnp.float32),
        mesh=mesh,
        scratch_types=scratch,
        compiler_params=pltpu.CompilerParams(use_tc_tiling_on_sc=False),
    )
    def k(x_hbm, seg_hbm, z_hbm, table_hbm, out_hbm, *sc):
        idx_v = sc[0:NBUF]
        rows_v = sc[NBUF:2 * NBUF]
        seg_v = sc[2 * NBUF]
        acc_sh = sc[2 * NBUF + 1]
        isem = sc[2 * NBUF + 2:2 * NBUF + 2 + NBUF]
        gsem = sc[2 * NBUF + 2 + NBUF:2 * NBUF + 2 + 2 * NBUF]
        ssem = sc[2 * NBUF + 2 + 2 * NBUF:2 * NBUF + 2 + 3 * NBUF]
        c = lax.axis_index("c")
        s = lax.axis_index("s")
        wid = c * NS + s
        base = s * ROWS_W

        def sctr_dst(t, j):
            return acc_sh.at[pl.ds(base + (t + j) * RPC, RPC)].at[seg_v]

        pltpu.sync_copy(z_hbm, acc_sh.at[pl.ds(base, ROWS_W)])
        pltpu.sync_copy(seg_hbm, seg_v)

        xbase = wid * IDX_W

        for j in range(NBUF):
            pltpu.async_copy(
                x_hbm.at[pl.ds(xbase + j * CHUNK, CHUNK)], idx_v[j], isem[j])
        for j in range(NBUF):
            pltpu.make_async_copy(
                x_hbm.at[pl.ds(xbase + j * CHUNK, CHUNK)], idx_v[j],
                isem[j]).wait()
            pltpu.async_copy(table_hbm.at[idx_v[j]], rows_v[j], gsem[j])

        @pl.loop(0, NCH, step=NBUF)
        def _(t):
            for j in range(NBUF):
                pltpu.make_async_copy(
                    table_hbm.at[idx_v[j]], rows_v[j], gsem[j]).wait()
                pltpu.async_copy(rows_v[j], sctr_dst(t, j), ssem[j],
                                 add=True)

                @pl.when(t + NBUF + j < NCH)
                def _():
                    pltpu.async_copy(
                        x_hbm.at[pl.ds(xbase + (t + NBUF + j) * CHUNK, CHUNK)],
                        idx_v[j], isem[j])

            for j in range(NBUF):
                @pl.when(t + NBUF + j < NCH)
                def _():
                    pltpu.make_async_copy(
                        rows_v[j], sctr_dst(t, j), ssem[j]).wait()
                    pltpu.make_async_copy(
                        x_hbm.at[pl.ds(xbase + (t + NBUF + j) * CHUNK, CHUNK)],
                        idx_v[j], isem[j]).wait()
                    pltpu.async_copy(
                        table_hbm.at[idx_v[j]], rows_v[j], gsem[j])

        for j in range(NBUF):
            pltpu.make_async_copy(
                rows_v[j], sctr_dst(NCH - NBUF, j), ssem[j]).wait()

        pltpu.sync_copy(
            acc_sh.at[pl.ds(base, ROWS_W)],
            out_hbm.at[pl.ds(wid * ROWS_W, ROWS_W)],
        )

    return k(x4, seg4, zeros, table)


def _tc_head(hsum, w, b2):
    blk = 2048

    def body(h_ref, w_ref, b_ref, o_ref):
        h = h_ref[...] * (1.0 / SEQ)
        logits = lax.dot_general(
            h, w_ref[...], (((1,), (1,)), ((), ())),
            preferred_element_type=jnp.float32,
            precision=lax.Precision.HIGHEST,
        )
        logits = logits + b_ref[...]
        m = jnp.max(logits, axis=-1, keepdims=True)
        e = jnp.exp(logits - m)
        ls = jnp.log(jnp.sum(e, axis=-1, keepdims=True)) + m
        o_ref[...] = logits - ls

    return pl.pallas_call(
        body,
        out_shape=jax.ShapeDtypeStruct((BATCH, OUT), jnp.float32),
        grid=(BATCH // blk,),
        in_specs=[
            pl.BlockSpec((blk, EMB), lambda i: (i, 0)),
            pl.BlockSpec((OUT, EMB), lambda i: (0, 0)),
            pl.BlockSpec((1, OUT), lambda i: (0, 0)),
        ],
        out_specs=pl.BlockSpec((blk, OUT), lambda i: (i, 0)),
    )(hsum, w, b2)


def kernel(X, table, W, b):
    sh = RBLK.bit_length() - 1
    xi = X.astype(jnp.int32).reshape(NW * NCH * CHUNK)
    x4 = (((xi >> sh) << sh) + ((xi & (RBLK // 4 - 1)) << 2)
          + ((xi >> (sh - 2)) & 3))
    seg4 = jnp.arange(CHUNK, dtype=jnp.int32) // SEQ
    zeros = jnp.zeros((ROWS_W, EMB), jnp.float32)
    table_r = _tc_repack_table(jnp.transpose(table))
    table_rm = table_r.reshape(VOCAB_PAD, EMB)
    hsum = _sc_pool_sums(x4, seg4, zeros, table_rm)
    return _tc_head(hsum, W, b.reshape(1, OUT))


# speedup vs baseline: 1.0310x; 1.0310x over previous
import functools

import jax
import jax.numpy as jnp
from jax import lax
from jax.experimental import pallas as pl
from jax.experimental.pallas import tpu as pltpu
from jax.experimental.pallas import tpu_sc as plsc

EMB = 32
OUT = 128
BATCH = 16384
SEQ = 200
VOCAB = 1000000
RBLK = 32768
NRBLK = -(-VOCAB // RBLK)
VOCAB_PAD = NRBLK * RBLK

NC = 2
NS = 16
NW = NC * NS
ROWS_W = BATCH // NW
IDX_W = ROWS_W * SEQ
CHUNK = 800
RPC = CHUNK // SEQ
NCH = IDX_W // CHUNK
NBUF = 4


def _tc_repack_table(tT):
    def body(t_ref, o_ref):
        x = t_ref[...]
        q = RBLK // 4
        stacked = jnp.concatenate(
            [x[:, j * q:(j + 1) * q] for j in range(4)], axis=0)
        o_ref[...] = jnp.transpose(stacked)

    return pl.pallas_call(
        body,
        out_shape=jax.ShapeDtypeStruct((VOCAB_PAD // 4, 128), jnp.float32),
        grid=(NRBLK,),
        in_specs=[pl.BlockSpec((EMB, RBLK), lambda i: (0, i))],
        out_specs=pl.BlockSpec((RBLK // 4, 128), lambda i: (i, 0)),
    )(tT)


def _sc_pool_sums(x4, seg4, zeros, table):
    mesh = plsc.VectorSubcoreMesh(core_axis_name="c", subcore_axis_name="s")

    scratch = (
        [pltpu.VMEM((CHUNK,), jnp.int32) for _ in range(NBUF)]
        + [pltpu.VMEM((CHUNK, EMB), jnp.float32) for _ in range(NBUF)]
        + [pltpu.VMEM((CHUNK,), jnp.int32)]
        + [pltpu.VMEM_SHARED((NS * ROWS_W, EMB), jnp.float32)]
        + [pltpu.SemaphoreType.DMA for _ in range(3 * NBUF)]
    )

    @functools.partial(
        pl.kernel,
        out_type=jax.ShapeDtypeStruct((BATCH, EMB), jnp.float32),
        mesh=mesh,
        scratch_types=scratch,
        compiler_params=pltpu.CompilerParams(use_tc_tiling_on_sc=False),
    )
    def k(x_hbm, seg_hbm, z_hbm, table_hbm, out_hbm, *sc):
        idx_v = sc[0:NBUF]
        rows_v = sc[NBUF:2 * NBUF]
        seg_v = sc[2 * NBUF]
        acc_sh = sc[2 * NBUF + 1]
        isem = sc[2 * NBUF + 2:2 * NBUF + 2 + NBUF]
        gsem = sc[2 * NBUF + 2 + NBUF:2 * NBUF + 2 + 2 * NBUF]
        ssem = sc[2 * NBUF + 2 + 2 * NBUF:2 * NBUF + 2 + 3 * NBUF]
        c = lax.axis_index("c")
        s = lax.axis_index("s")
        wid = c * NS + s
        base = s * ROWS_W

        def sctr_dst(t, j):
            return acc_sh.at[pl.ds(base + (t + j) * RPC, RPC)].at[seg_v]

        pltpu.sync_copy(z_hbm, acc_sh.at[pl.ds(base, ROWS_W)])
        pltpu.sync_copy(seg_hbm, seg_v)

        xbase = wid * IDX_W

        for j in range(NBUF):
            pltpu.async_copy(
                x_hbm.at[pl.ds(xbase + j * CHUNK, CHUNK)], idx_v[j], isem[j])
        for j in range(NBUF):
            pltpu.make_async_copy(
                x_hbm.at[pl.ds(xbase + j * CHUNK, CHUNK)], idx_v[j],
                isem[j]).wait()
            pltpu.async_copy(table_hbm.at[idx_v[j]], rows_v[j], gsem[j])

        @pl.loop(0, NCH, step=NBUF)
        def _(t):
            for j in range(NBUF):
                pltpu.make_async_copy(
                    table_hbm.at[idx_v[j]], rows_v[j], gsem[j]).wait()
                pltpu.async_copy(rows_v[j], sctr_dst(t, j), ssem[j],
                                 add=True)

                @pl.when(t + NBUF + j < NCH)
                def _():
                    pltpu.async_copy(
                        x_hbm.at[pl.ds(xbase + (t + NBUF + j) * CHUNK, CHUNK)],
                        idx_v[j], isem[j])

            for j in range(NBUF):
                @pl.when(t + NBUF + j < NCH)
                def _():
                    pltpu.make_async_copy(
                        rows_v[j], sctr_dst(t, j), ssem[j]).wait()
                    pltpu.make_async_copy(
                        x_hbm.at[pl.ds(xbase + (t + NBUF + j) * CHUNK, CHUNK)],
                        idx_v[j], isem[j]).wait()
                    pltpu.async_copy(
                        table_hbm.at[idx_v[j]], rows_v[j], gsem[j])

        for j in range(NBUF):
            pltpu.make_async_copy(
                rows_v[j], sctr_dst(NCH - NBUF, j), ssem[j]).wait()

        pltpu.sync_copy(
            acc_sh.at[pl.ds(base, ROWS_W)],
            out_hbm.at[pl.ds(wid * ROWS_W, ROWS_W)],
        )

    return k(x4, seg4, zeros, table)


def _tc_head(hsum, w, b2):
    blk = 2048

    def body(h_ref, w_ref, b_ref, o_ref):
        h = h_ref[...] * (1.0 / SEQ)
        logits = lax.dot_general(
            h, w_ref[...], (((1,), (1,)), ((), ())),
            preferred_element_type=jnp.float32,
            precision=lax.Precision.HIGHEST,
        )
        logits = logits + b_ref[...]
        m = jnp.max(logits, axis=-1, keepdims=True)
        e = jnp.exp(logits - m)
        ls = jnp.log(jnp.sum(e, axis=-1, keepdims=True)) + m
        o_ref[...] = logits - ls

    return pl.pallas_call(
        body,
        out_shape=jax.ShapeDtypeStruct((BATCH, OUT), jnp.float32),
        grid=(BATCH // blk,),
        in_specs=[
            pl.BlockSpec((blk, EMB), lambda i: (i, 0)),
            pl.BlockSpec((OUT, EMB), lambda i: (0, 0)),
            pl.BlockSpec((1, OUT), lambda i: (0, 0)),
        ],
        out_specs=pl.BlockSpec((blk, OUT), lambda i: (i, 0)),
    )(hsum, w, b2)


def kernel(X, table, W, b):
    sh = RBLK.bit_length() - 1
    xi = X.astype(jnp.int32).reshape(NW * NCH * CHUNK)
    x4 = (((xi >> sh) << sh) + ((xi & (RBLK // 4 - 1)) << 2)
          + ((xi >> (sh - 2)) & 3))
    seg4 = jnp.arange(CHUNK, dtype=jnp.int32) // SEQ
    zeros = jnp.zeros((ROWS_W, EMB), jnp.float32)
    table_r = _tc_repack_table(jnp.transpose(table))
    table_rm = table_r.reshape(VOCAB_PAD, EMB)
    hsum = _sc_pool_sums(x4, seg4, zeros, table_rm)
    return _tc_head(hsum, W, b.reshape(1, OUT))


# speedup vs baseline: 1.0352x; 1.0041x over previous
import functools

import jax
import jax.numpy as jnp
from jax import lax
from jax.experimental import pallas as pl
from jax.experimental.pallas import tpu as pltpu
from jax.experimental.pallas import tpu_sc as plsc

EMB = 32
OUT = 128
BATCH = 16384
SEQ = 200
VOCAB = 1000000
RBLK = 65536
NRBLK = -(-VOCAB // RBLK)
VOCAB_PAD = NRBLK * RBLK

NC = 2
NS = 16
NW = NC * NS
ROWS_W = BATCH // NW
IDX_W = ROWS_W * SEQ
CHUNK = 800
RPC = CHUNK // SEQ
NCH = IDX_W // CHUNK
NBUF = 4


def _tc_repack_table(tT):
    def body(t_ref, o_ref):
        x = t_ref[...]
        q = RBLK // 4
        stacked = jnp.concatenate(
            [x[:, j * q:(j + 1) * q] for j in range(4)], axis=0)
        o_ref[...] = jnp.transpose(stacked)

    return pl.pallas_call(
        body,
        out_shape=jax.ShapeDtypeStruct((VOCAB_PAD // 4, 128), jnp.float32),
        grid=(NRBLK,),
        in_specs=[pl.BlockSpec((EMB, RBLK), lambda i: (0, i))],
        out_specs=pl.BlockSpec((RBLK // 4, 128), lambda i: (i, 0)),
    )(tT)


def _sc_pool_sums(x4, seg4, zeros, table):
    mesh = plsc.VectorSubcoreMesh(core_axis_name="c", subcore_axis_name="s")

    scratch = (
        [pltpu.VMEM((CHUNK,), jnp.int32) for _ in range(NBUF)]
        + [pltpu.VMEM((CHUNK, EMB), jnp.float32) for _ in range(NBUF)]
        + [pltpu.VMEM((CHUNK,), jnp.int32)]
        + [pltpu.VMEM_SHARED((NS * ROWS_W, EMB), jnp.float32)]
        + [pltpu.SemaphoreType.DMA for _ in range(3 * NBUF)]
    )

    @functools.partial(
        pl.kernel,
        out_type=jax.ShapeDtypeStruct((BATCH, EMB), jnp.float32),
        mesh=mesh,
        scratch_types=scratch,
        compiler_params=pltpu.CompilerParams(use_tc_tiling_on_sc=False),
    )
    def k(x_hbm, seg_hbm, z_hbm, table_hbm, out_hbm, *sc):
        idx_v = sc[0:NBUF]
        rows_v = sc[NBUF:2 * NBUF]
        seg_v = sc[2 * NBUF]
        acc_sh = sc[2 * NBUF + 1]
        isem = sc[2 * NBUF + 2:2 * NBUF + 2 + NBUF]
        gsem = sc[2 * NBUF + 2 + NBUF:2 * NBUF + 2 + 2 * NBUF]
        ssem = sc[2 * NBUF + 2 + 2 * NBUF:2 * NBUF + 2 + 3 * NBUF]
        c = lax.axis_index("c")
        s = lax.axis_index("s")
        wid = c * NS + s
        base = s * ROWS_W

        def sctr_dst(t, j):
            return acc_sh.at[pl.ds(base + (t + j) * RPC, RPC)].at[seg_v]

        pltpu.sync_copy(z_hbm, acc_sh.at[pl.ds(base, ROWS_W)])
        pltpu.sync_copy(seg_hbm, seg_v)

        xbase = wid * IDX_W

        for j in range(NBUF):
            pltpu.async_copy(
                x_hbm.at[pl.ds(xbase + j * CHUNK, CHUNK)], idx_v[j], isem[j])
        for j in range(NBUF):
            pltpu.make_async_copy(
                x_hbm.at[pl.ds(xbase + j * CHUNK, CHUNK)], idx_v[j],
                isem[j]).wait()
            pltpu.async_copy(table_hbm.at[idx_v[j]], rows_v[j], gsem[j])

        @pl.loop(0, NCH, step=NBUF)
        def _(t):
            for j in range(NBUF):
                pltpu.make_async_copy(
                    table_hbm.at[idx_v[j]], rows_v[j], gsem[j]).wait()
                pltpu.async_copy(rows_v[j], sctr_dst(t, j), ssem[j],
                                 add=True)

                @pl.when(t + NBUF + j < NCH)
                def _():
                    pltpu.async_copy(
                        x_hbm.at[pl.ds(xbase + (t + NBUF + j) * CHUNK, CHUNK)],
                        idx_v[j], isem[j])

            for j in range(NBUF):
                @pl.when(t + NBUF + j < NCH)
                def _():
                    pltpu.make_async_copy(
                        rows_v[j], sctr_dst(t, j), ssem[j]).wait()
                    pltpu.make_async_copy(
                        x_hbm.at[pl.ds(xbase + (t + NBUF + j) * CHUNK, CHUNK)],
                        idx_v[j], isem[j]).wait()
                    pltpu.async_copy(
                        table_hbm.at[idx_v[j]], rows_v[j], gsem[j])

        for j in range(NBUF):
            pltpu.make_async_copy(
                rows_v[j], sctr_dst(NCH - NBUF, j), ssem[j]).wait()

        pltpu.sync_copy(
            acc_sh.at[pl.ds(base, ROWS_W)],
            out_hbm.at[pl.ds(wid * ROWS_W, ROWS_W)],
        )

    return k(x4, seg4, zeros, table)


def _tc_head(hsum, w, b2):
    blk = 2048

    def body(h_ref, w_ref, b_ref, o_ref):
        h = h_ref[...] * (1.0 / SEQ)
        logits = lax.dot_general(
            h, w_ref[...], (((1,), (1,)), ((), ())),
            preferred_element_type=jnp.float32,
            precision=lax.Precision.HIGHEST,
        )
        logits = logits + b_ref[...]
        m = jnp.max(logits, axis=-1, keepdims=True)
        e = jnp.exp(logits - m)
        ls = jnp.log(jnp.sum(e, axis=-1, keepdims=True)) + m
        o_ref[...] = logits - ls

    return pl.pallas_call(
        body,
        out_shape=jax.ShapeDtypeStruct((BATCH, OUT), jnp.float32),
        grid=(BATCH // blk,),
        in_specs=[
            pl.BlockSpec((blk, EMB), lambda i: (i, 0)),
            pl.BlockSpec((OUT, EMB), lambda i: (0, 0)),
            pl.BlockSpec((1, OUT), lambda i: (0, 0)),
        ],
        out_specs=pl.BlockSpec((blk, OUT), lambda i: (i, 0)),
    )(hsum, w, b2)


def kernel(X, table, W, b):
    sh = RBLK.bit_length() - 1
    xi = X.astype(jnp.int32).reshape(NW * NCH * CHUNK)
    x4 = (((xi >> sh) << sh) + ((xi & (RBLK // 4 - 1)) << 2)
          + ((xi >> (sh - 2)) & 3))
    seg4 = jnp.arange(CHUNK, dtype=jnp.int32) // SEQ
    zeros = jnp.zeros((ROWS_W, EMB), jnp.float32)
    table_r = _tc_repack_table(jnp.transpose(table))
    table_rm = table_r.reshape(VOCAB_PAD, EMB)
    hsum = _sc_pool_sums(x4, seg4, zeros, table_rm)
    return _tc_head(hsum, W, b.reshape(1, OUT))
